# baseline (speedup 1.0000x reference)
import jax
import jax.numpy as jnp
from jax import lax
from jax.experimental import pallas as pl
from jax.experimental.pallas import tpu as pltpu, tpu_sc as plsc

B = 1024
L = 200
V = 100001
V8 = 100008
NC = 2
NS = 16
NW = NC * NS
UPW = B // NW
NEVAL = 60000
NV = 13


def _full(c):
    return jnp.full((16,), c, jnp.int32)


def _off(j):
    return j * 16 if j < 12 else 184


def _body(log_ref, t1_ref, t2_ref, user_ref, mt_ref, wt_ref, we_ref,
          out_ref, itemv, t1v, t2v, midx, widx, ridx, uloc, rbuf,
          mkbuf, wkbuf, otile, m_sem, w_sem, r_sem):
    wid = lax.axis_index("s") * NC + lax.axis_index("c")
    iota = lax.iota(jnp.int32, 16)

    pltpu.sync_copy(user_ref.at[pl.ds(wid * UPW, UPW)], uloc)
    for i in range(2):
        u = uloc[pl.ds(i * 16, 16)]
        r = (u - 1) * 6
        r = jnp.where(r < 0, r + NEVAL, r)
        for k in range(6):
            plsc.store_scatter(ridx, [iota * 6 + (i * 96 + k)], r + k)
    c0 = pltpu.async_copy(we_ref.at[ridx.at[pl.ds(0, 96)]],
                          rbuf.at[pl.ds(0, 96)], r_sem)
    c1 = pltpu.async_copy(we_ref.at[ridx.at[pl.ds(96, 96)]],
                          rbuf.at[pl.ds(96, 96)], r_sem)
    c0.wait()
    c1.wait()

    def group(g, _):
        b = wid * UPW + g
        pltpu.sync_copy(log_ref.at[b], itemv.at[pl.ds(0, L)])
        pltpu.sync_copy(t1_ref.at[b], t1v.at[pl.ds(0, L)])
        pltpu.sync_copy(t2_ref.at[b], t2v.at[pl.ds(0, L)])
        for j in range(NV):
            o = _off(j)
            it = itemv[pl.ds(o, 16)]
            mb = t1v[pl.ds(o, 16)] * (12 * V8) + it
            wb = t2v[pl.ds(o, 16)] * (6 * V8) + it
            for k in range(12):
                midx[pl.ds(k * L + o, 16)] = mb + k * V8
            for k in range(6):
                widx[pl.ds(k * L + o, 16)] = wb + k * V8
        cs = []
        for k in range(12):
            cs.append(pltpu.async_copy(
                mt_ref.at[midx.at[pl.ds(k * L, 128)]],
                mkbuf.at[pl.ds(k * L, 128)], m_sem))
            cs.append(pltpu.async_copy(
                mt_ref.at[midx.at[pl.ds(k * L + 128, 72)]],
                mkbuf.at[pl.ds(k * L + 128, 72)], m_sem))
        for k in range(6):
            cs.append(pltpu.async_copy(
                wt_ref.at[widx.at[pl.ds(k * L, 128)]],
                wkbuf.at[pl.ds(k * L, 128)], w_sem))
            cs.append(pltpu.async_copy(
                wt_ref.at[widx.at[pl.ds(k * L + 128, 72)]],
                wkbuf.at[pl.ds(k * L + 128, 72)], w_sem))
        for j in range(NV):
            o = _off(j)
            rows = o + iota
            for k in range(6):
                v = rbuf[g * 6 + k, pl.ds(o, 16)]
                plsc.store_scatter(otile, [rows, _full(18 + k)], v)
        for c in cs:
            c.wait()
        for j in range(NV):
            o = _off(j)
            rows = o + iota
            for k in range(12):
                plsc.store_scatter(otile, [rows, _full(k)],
                                   mkbuf[pl.ds(k * L + o, 16)])
            for k in range(6):
                plsc.store_scatter(otile, [rows, _full(12 + k)],
                                   wkbuf[pl.ds(k * L + o, 16)])
        pltpu.sync_copy(otile, out_ref.at[b])
        return 0

    lax.fori_loop(0, UPW, group, 0)


@jax.jit
def _run(log2d, t12d, t22d, user, mt_flat, wt_flat, we):
    mesh = plsc.VectorSubcoreMesh(core_axis_name="c", subcore_axis_name="s")
    f = pl.kernel(
        _body,
        out_type=jax.ShapeDtypeStruct((B, L, 24), jnp.float32),
        mesh=mesh,
        compiler_params=pltpu.CompilerParams(
            needs_layout_passes=False, use_tc_tiling_on_sc=False),
        scratch_types=[
            pltpu.VMEM((208,), jnp.int32),
            pltpu.VMEM((208,), jnp.int32),
            pltpu.VMEM((208,), jnp.int32),
            pltpu.VMEM((12 * L,), jnp.int32),
            pltpu.VMEM((6 * L,), jnp.int32),
            pltpu.VMEM((192,), jnp.int32),
            pltpu.VMEM((UPW,), jnp.int32),
            pltpu.VMEM((UPW * 6, L), jnp.float32),
            pltpu.VMEM((12 * L,), jnp.float32),
            pltpu.VMEM((6 * L,), jnp.float32),
            pltpu.VMEM((L, 24), jnp.float32),
            pltpu.SemaphoreType.DMA,
            pltpu.SemaphoreType.DMA,
            pltpu.SemaphoreType.DMA,
        ],
    )
    return f(log2d, t12d, t22d, user, mt_flat, wt_flat, we)


def kernel(log_seqs, time1_seqs, time2_seqs, user, month_pop_table,
           week_pop_table, week_eval_pop):
    mt_flat = jnp.pad(month_pop_table, ((0, 0), (0, V8 - V))).reshape(-1)
    wt_flat = jnp.pad(week_pop_table, ((0, 0), (0, V8 - V))).reshape(-1)
    out = _run(
        log_seqs.astype(jnp.int32),
        time1_seqs.astype(jnp.int32),
        time2_seqs.astype(jnp.int32),
        user.astype(jnp.int32),
        mt_flat, wt_flat, week_eval_pop)
    return lax.stop_gradient(out)
